# Initial kernel scaffold; baseline (speedup 1.0000x reference)
import jax
import jax.numpy as jnp
from jax.experimental import pallas as pl


def kernel(x, table):
    raise NotImplementedError("write your pallas kernel here")


# speedup vs baseline: 1.8694x; 1.8694x over previous
import jax
import jax.numpy as jnp
from jax import lax
from jax.experimental import pallas as pl
from jax.experimental.pallas import tpu as pltpu
from jax.experimental.pallas import tpu_sc as plsc

_BATCH = 16384
_HIST = 50
_D = 64
_B = _BATCH * _HIST
_NC, _NS = 2, 16
_NW = _NC * _NS
_BPW = _B // _NW
_C = 128
_NCHUNK = _BPW // _C
_NBUF = 4
_NGRP = _NCHUNK // _NBUF


def _body(table, idx, out, idx_v, rows_v, *sems):
    sem_g = sems[:_NBUF]
    sem_s = sems[_NBUF:]
    wid = lax.axis_index("s") * _NC + lax.axis_index("c")
    base = wid * _BPW

    pltpu.sync_copy(idx.at[wid], idx_v)

    def start_gather(g, b):
        pltpu.async_copy(table.at[idx_v.at[g]], rows_v.at[b], sem_g[b])

    def wait_gather(g, b):
        pltpu.make_async_copy(table.at[idx_v.at[g]], rows_v.at[b],
                              sem_g[b]).wait()

    def start_store(g, b):
        pltpu.async_copy(rows_v.at[b], out.at[pl.ds(base + g * _C, _C)],
                         sem_s[b])

    def wait_store(g, b):
        pltpu.make_async_copy(rows_v.at[b], out.at[pl.ds(base + g * _C, _C)],
                              sem_s[b]).wait()

    for b in range(_NBUF):
        start_gather(b, b)

    def group(grp, carry):
        g0 = grp * _NBUF
        for b in range(_NBUF):
            wait_gather(g0 + b, b)
            start_store(g0 + b, b)
        for b in range(_NBUF):
            wait_store(g0 + b, b)

            @pl.when(grp < _NGRP - 1)
            def _issue(b=b):
                start_gather(g0 + _NBUF + b, b)

        return carry

    lax.fori_loop(0, _NGRP, group, None)


def kernel(x, table):
    idx = x.reshape(_NW, _NCHUNK, _C).astype(jnp.int32)
    mesh = plsc.VectorSubcoreMesh(
        core_axis_name="c", subcore_axis_name="s",
        num_cores=_NC, num_subcores=_NS)
    f = pl.kernel(
        _body,
        out_type=jax.ShapeDtypeStruct((_B, _D), jnp.float32),
        mesh=mesh,
        scratch_types=[
            pltpu.VMEM((_NCHUNK, _C), jnp.int32),
            pltpu.VMEM((_NBUF, _C, _D), jnp.float32),
        ] + [pltpu.SemaphoreType.DMA] * (2 * _NBUF),
        compiler_params=pltpu.CompilerParams(use_tc_tiling_on_sc=False),
    )
    out = f(table, idx)
    return out.reshape(_BATCH, _HIST, _D)


# speedup vs baseline: 1.8701x; 1.0004x over previous
import jax
import jax.numpy as jnp
from jax import lax
from jax.experimental import pallas as pl
from jax.experimental.pallas import tpu as pltpu
from jax.experimental.pallas import tpu_sc as plsc

_BATCH = 16384
_HIST = 50
_D = 64
_B = _BATCH * _HIST
_NC, _NS = 2, 16
_NW = _NC * _NS
_BPW = _B // _NW
_C = 256
_NCHUNK = _BPW // _C
_NBUF = 4
_NGRP = _NCHUNK // _NBUF


def _body(table, idx, out, idx_v, rows_v, *sems):
    sem_g = sems[:_NBUF]
    sem_s = sems[_NBUF:]
    wid = lax.axis_index("s") * _NC + lax.axis_index("c")
    base = wid * _BPW

    pltpu.sync_copy(idx.at[wid], idx_v)

    def start_gather(g, b):
        pltpu.async_copy(table.at[idx_v.at[g]], rows_v.at[b], sem_g[b])

    def wait_gather(g, b):
        pltpu.make_async_copy(table.at[idx_v.at[g]], rows_v.at[b],
                              sem_g[b]).wait()

    def start_store(g, b):
        pltpu.async_copy(rows_v.at[b], out.at[pl.ds(base + g * _C, _C)],
                         sem_s[b])

    def wait_store(g, b):
        pltpu.make_async_copy(rows_v.at[b], out.at[pl.ds(base + g * _C, _C)],
                              sem_s[b]).wait()

    for b in range(_NBUF):
        start_gather(b, b)

    def group(grp, carry):
        g0 = grp * _NBUF
        for b in range(_NBUF):
            wait_gather(g0 + b, b)
            start_store(g0 + b, b)
        for b in range(_NBUF):
            wait_store(g0 + b, b)

            @pl.when(grp < _NGRP - 1)
            def _issue(b=b):
                start_gather(g0 + _NBUF + b, b)

        return carry

    lax.fori_loop(0, _NGRP, group, None)


def kernel(x, table):
    idx = x.reshape(_NW, _NCHUNK, _C).astype(jnp.int32)
    mesh = plsc.VectorSubcoreMesh(
        core_axis_name="c", subcore_axis_name="s",
        num_cores=_NC, num_subcores=_NS)
    f = pl.kernel(
        _body,
        out_type=jax.ShapeDtypeStruct((_B, _D), jnp.float32),
        mesh=mesh,
        scratch_types=[
            pltpu.VMEM((_NCHUNK, _C), jnp.int32),
            pltpu.VMEM((_NBUF, _C, _D), jnp.float32),
        ] + [pltpu.SemaphoreType.DMA] * (2 * _NBUF),
        compiler_params=pltpu.CompilerParams(use_tc_tiling_on_sc=False),
    )
    out = f(table, idx)
    return out.reshape(_BATCH, _HIST, _D)
